# Initial kernel scaffold; baseline (speedup 1.0000x reference)
import jax
import jax.numpy as jnp
from jax.experimental import pallas as pl


def kernel(x, segment_ids, W_enc, b_enc, W_q0, W_q1):
    raise NotImplementedError("write your pallas kernel here")


# speedup vs baseline: 27.6955x; 27.6955x over previous
import jax
import jax.numpy as jnp
from jax import lax
from jax.experimental import pallas as pl
from jax.experimental.pallas import tpu as pltpu

_TOTAL = 32768
_B = 16
_D = 128
_BLK = 4096
_NBLK = _TOTAL // _BLK


def _body(x_ref, seg_ref, wenc_ref, benc_ref, wq0_ref, wq1_ref,
          keys_ref, q0_ref, q1_ref, acc_ref, cnt_ref):
    i = pl.program_id(0)

    @pl.when(i == 0)
    def _init():
        acc_ref[...] = jnp.zeros((_B, _D), jnp.float32)
        cnt_ref[...] = jnp.zeros((_B, _D), jnp.float32)

    seg = seg_ref[0]
    onehot = (seg[:, :, None] == lax.broadcasted_iota(jnp.int32, (1, _BLK, _B), 2)
              ).astype(jnp.float32)[0]
    x = x_ref[...]
    acc_ref[...] += lax.dot_general(onehot, x, (((0,), (0,)), ((), ())),
                                    preferred_element_type=jnp.float32)
    cnt_ref[...] += lax.dot_general(onehot, jnp.ones((_BLK, _D), jnp.float32),
                                    (((0,), (0,)), ((), ())),
                                    preferred_element_type=jnp.float32)

    @pl.when(i == _NBLK - 1)
    def _finish():
        s = acc_ref[...]
        cnt = cnt_ref[...]
        denom = jnp.maximum(cnt, 1.0)
        keys = (jnp.dot(s, wenc_ref[...], preferred_element_type=jnp.float32)
                + cnt * benc_ref[...]) / denom
        keys_ref[...] = keys
        q0_ref[...] = jnp.dot(keys, wq0_ref[...], preferred_element_type=jnp.float32)
        q1_ref[...] = jnp.dot(keys, wq1_ref[...], preferred_element_type=jnp.float32)


def kernel(x, segment_ids, W_enc, b_enc, W_q0, W_q1):
    seg3 = segment_ids.reshape(_NBLK, 1, _BLK)
    benc2 = b_enc.reshape(1, _D)
    out_shape = [jax.ShapeDtypeStruct((_B, _D), jnp.float32)] * 3
    keys, q0, q1 = pl.pallas_call(
        _body,
        grid=(_NBLK,),
        in_specs=[
            pl.BlockSpec((_BLK, _D), lambda i: (i, 0)),
            pl.BlockSpec((1, 1, _BLK), lambda i: (i, 0, 0)),
            pl.BlockSpec((_D, _D), lambda i: (0, 0)),
            pl.BlockSpec((1, _D), lambda i: (0, 0)),
            pl.BlockSpec((_D, _D), lambda i: (0, 0)),
            pl.BlockSpec((_D, _D), lambda i: (0, 0)),
        ],
        out_specs=[pl.BlockSpec((_B, _D), lambda i: (0, 0))] * 3,
        out_shape=out_shape,
        scratch_shapes=[pltpu.VMEM((_B, _D), jnp.float32),
                        pltpu.VMEM((_B, _D), jnp.float32)],
        compiler_params=pltpu.CompilerParams(
            dimension_semantics=("arbitrary",)),
    )(x, seg3, W_enc, benc2, W_q0, W_q1)
    return (keys, q0, q1)
